# baseline (speedup 1.0000x reference)
import functools

import jax
import jax.numpy as jnp
from jax import lax
from jax.experimental import pallas as pl
from jax.experimental.pallas import tpu as pltpu
from jax.experimental.pallas import tpu_sc as plsc

N = 100000
D = 128
B = 1024

NT = 100000
NS = N - NT

R = 1000
NBT = NT // R

NW = 32
RPW = NS // NW
C = 112
SPAN = 1792
NCH = SPAN // C
NJ = D // 16

_mesh = plsc.VectorSubcoreMesh(core_axis_name="c", subcore_axis_name="s")


def _allsum16(v):
    idx = lax.broadcasted_iota(jnp.int32, (16,), 0)
    dn = lax.GatherDimensionNumbers(
        offset_dims=(), collapsed_slice_dims=(0,), start_index_map=(0,))
    for sh in (1, 2, 4, 8):
        perm = jnp.bitwise_xor(idx, sh)
        v = v + lax.gather(v, perm.reshape(16, 1), dn, (1,),
                           mode=lax.GatherScatterMode.PROMISE_IN_BOUNDS)
    return v


@functools.partial(
    pl.kernel,
    mesh=_mesh,
    out_type=jax.ShapeDtypeStruct((2, B, D), jnp.float32),
    scratch_types=[
        pltpu.VMEM((C, D), jnp.float32),
        pltpu.VMEM((C, D), jnp.float32),
        pltpu.VMEM((C, D), jnp.float32),
        pltpu.VMEM((C, D), jnp.float32),
        pltpu.VMEM((NCH, C), jnp.int32),
        pltpu.VMEM((D,), jnp.float32),
        pltpu.VMEM((16,), jnp.float32),
        pltpu.VMEM_SHARED((B, D), jnp.float32),
        pltpu.SemaphoreType.DMA,
        pltpu.SemaphoreType.DMA,
        pltpu.SemaphoreType.DMA,
        pltpu.SemaphoreType.DMA,
        pltpu.SemaphoreType.DMA,
    ],
)
def _sc_weight_sum(feats_hbm, seg_hbm, w_hbm, b_hbm, out_hbm,
                   fbuf0, fbuf1, sbuf0, sbuf1, idbuf, wbuf, bbuf, acc,
                   rsem0, rsem1, ssem0, ssem1, isem):
    cid = lax.axis_index("c")
    sid = lax.axis_index("s")
    wid = sid * 2 + cid
    lo = NT + wid * RPW
    hi = lo + RPW
    base8 = jnp.minimum((lo // 8) * 8, N - SPAN)

    pltpu.sync_copy(w_hbm, wbuf)
    pltpu.sync_copy(b_hbm, bbuf)

    def idstart(k, carry):
        pltpu.async_copy(seg_hbm.at[pl.ds(base8 + k * C, C)], idbuf.at[k], isem)
        return carry

    lax.fori_loop(0, NCH, idstart, 0)

    def iddrain(k, carry):
        pltpu.make_async_copy(
            seg_hbm.at[pl.ds(base8 + k * C, C)], idbuf.at[k], isem).wait()
        return carry

    lax.fori_loop(0, NCH, iddrain, 0)

    zrows = B // 16
    zero = jnp.zeros((16,), jnp.float32)

    def zbody(r, carry):
        for j in range(NJ):
            sbuf0[r, pl.ds(16 * j, 16)] = zero
        return carry

    lax.fori_loop(0, zrows, zbody, 0)
    pltpu.sync_copy(sbuf0.at[pl.ds(0, zrows)], acc.at[pl.ds(sid * zrows, zrows)])
    plsc.subcore_barrier()

    wv = tuple(wbuf[pl.ds(16 * j, 16)] for j in range(NJ))
    bv = bbuf[...]
    fbufs = (fbuf0, fbuf1)
    sbufs = (sbuf0, sbuf1)
    rsems = (rsem0, rsem1)
    ssems = (ssem0, ssem1)

    def _feats_slice(k):
        return feats_hbm.at[pl.ds(base8 + k * C, C)]

    pltpu.async_copy(_feats_slice(0), fbuf0, rsem0)
    pltpu.async_copy(_feats_slice(1), fbuf1, rsem1)

    def outer(kk, carry):
        for b in (0, 1):
            k = 2 * kk + b
            fb, sb, rs, ss = fbufs[b], sbufs[b], rsems[b], ssems[b]
            pltpu.make_async_copy(_feats_slice(k), fb, rs).wait()

            @pl.when(k >= 2)
            def _wait_prev_scatter():
                pltpu.make_async_copy(sb, acc.at[idbuf.at[k - 2]], ss).wait()

            @plsc.parallel_loop(0, C, unroll=2)
            def row_body(r):
                g = base8 + k * C + r
                valid = jnp.logical_and(g >= lo, g < hi)
                f = [fb[r, pl.ds(16 * j, 16)] for j in range(NJ)]
                p = f[0] * wv[0]
                for j in range(1, NJ):
                    p = p + f[j] * wv[j]
                sv = _allsum16(p) + bv
                m = jnp.where(valid, 1.0, 0.0).astype(jnp.float32)
                w16 = jnp.full((16,), m, jnp.float32) / (1.0 + jnp.exp(-sv))
                for j in range(NJ):
                    sb[r, pl.ds(16 * j, 16)] = f[j] * w16

            @pl.when(k + 2 < NCH)
            def _prefetch_next():
                pltpu.async_copy(_feats_slice(k + 2), fb, rs)

            pltpu.async_copy(sb, acc.at[idbuf.at[k]], ss, add=True)
        return carry

    lax.fori_loop(0, NCH // 2, outer, 0)

    pltpu.make_async_copy(sbuf0, acc.at[idbuf.at[NCH - 2]], ssem0).wait()
    pltpu.make_async_copy(sbuf1, acc.at[idbuf.at[NCH - 1]], ssem1).wait()

    plsc.subcore_barrier()
    pltpu.sync_copy(acc.at[pl.ds(sid * zrows, zrows)],
                    out_hbm.at[cid, pl.ds(sid * zrows, zrows)])


WK = 128


def _tc_body(seg_ref, feats_ref, W_ref, b_ref, out_ref):
    i = pl.program_id(0)

    @pl.when(i == 0)
    def _init():
        out_ref[...] = jnp.zeros_like(out_ref)

    f = feats_ref[...]
    y = jnp.dot(f, W_ref[...], preferred_element_type=jnp.float32) + b_ref[0, 0]
    w = 1.0 / (1.0 + jnp.exp(-y))
    weighted = (f * w).astype(jnp.bfloat16)
    seg = seg_ref[0, 0, :]
    base = jnp.minimum((seg[0] // 8) * 8, B - WK)
    span_ok = seg[R - 1] - base < WK

    @pl.when(span_ok)
    def _windowed():
        onehot = (
            seg[None, :] - base
            == jax.lax.broadcasted_iota(jnp.int32, (WK, R), 0)
        ).astype(jnp.bfloat16)
        out_ref[pl.ds(base, WK), :] += jnp.dot(
            onehot, weighted, preferred_element_type=jnp.float32)

    @pl.when(jnp.logical_not(span_ok))
    def _full():
        onehot = (
            seg[None, :] == jax.lax.broadcasted_iota(jnp.int32, (B, R), 0)
        ).astype(jnp.bfloat16)
        out_ref[...] += jnp.dot(
            onehot, weighted, preferred_element_type=jnp.float32)


def _tc_partial(feats, seg, W, b):
    seg3 = seg.reshape(N // R, 1, R)
    b2 = b.reshape(1, 1).astype(jnp.float32)
    return pl.pallas_call(
        _tc_body,
        grid=(NBT,),
        in_specs=[
            pl.BlockSpec((1, 1, R), lambda i: (i, 0, 0)),
            pl.BlockSpec((R, D), lambda i: (i, 0)),
            pl.BlockSpec((D, 1), lambda i: (0, 0)),
            pl.BlockSpec((1, 1), lambda i: (0, 0)),
        ],
        out_specs=pl.BlockSpec((B, D), lambda i: (0, 0)),
        out_shape=jax.ShapeDtypeStruct((B, D), jnp.float32),
    )(seg3, feats, W, b2)


def _combine_body(tc_ref, p_ref, out_ref):
    out_ref[...] = tc_ref[...] + p_ref[0] + p_ref[1]


def _combine(tc_part, sc_partials):
    return pl.pallas_call(
        _combine_body,
        out_shape=jax.ShapeDtypeStruct((B, D), jnp.float32),
    )(tc_part, sc_partials)


def kernel(feats, segment_ids, W, b):
    seg = segment_ids.astype(jnp.int32)
    return _tc_partial(feats, seg, W, b)


# speedup vs baseline: 1.6595x; 1.6595x over previous
import functools

import jax
import jax.numpy as jnp
from jax import lax
from jax.experimental import pallas as pl
from jax.experimental.pallas import tpu as pltpu
from jax.experimental.pallas import tpu_sc as plsc

N = 100000
D = 128
B = 1024

NT = 36000
NS = N - NT

R = 1000
NBT = NT // R

NW = 32
RPW = NS // NW
C = 112
SPAN = 2016
NCH = SPAN // C
NJ = D // 16

_mesh = plsc.VectorSubcoreMesh(core_axis_name="c", subcore_axis_name="s")


def _allsum16(v):
    idx = lax.broadcasted_iota(jnp.int32, (16,), 0)
    dn = lax.GatherDimensionNumbers(
        offset_dims=(), collapsed_slice_dims=(0,), start_index_map=(0,))
    for sh in (1, 2, 4, 8):
        perm = jnp.bitwise_xor(idx, sh)
        v = v + lax.gather(v, perm.reshape(16, 1), dn, (1,),
                           mode=lax.GatherScatterMode.PROMISE_IN_BOUNDS)
    return v


@functools.partial(
    pl.kernel,
    mesh=_mesh,
    out_type=jax.ShapeDtypeStruct((2, B, D), jnp.float32),
    scratch_types=[
        pltpu.VMEM((C, D), jnp.float32),
        pltpu.VMEM((C, D), jnp.float32),
        pltpu.VMEM((C, D), jnp.float32),
        pltpu.VMEM((C, D), jnp.float32),
        pltpu.VMEM((NCH, C), jnp.int32),
        pltpu.VMEM((D,), jnp.float32),
        pltpu.VMEM((16,), jnp.float32),
        pltpu.VMEM_SHARED((B, D), jnp.float32),
        pltpu.SemaphoreType.DMA,
        pltpu.SemaphoreType.DMA,
        pltpu.SemaphoreType.DMA,
        pltpu.SemaphoreType.DMA,
        pltpu.SemaphoreType.DMA,
    ],
)
def _sc_weight_sum(feats_hbm, seg_hbm, w_hbm, b_hbm, out_hbm,
                   fbuf0, fbuf1, sbuf0, sbuf1, idbuf, wbuf, bbuf, acc,
                   rsem0, rsem1, ssem0, ssem1, isem):
    cid = lax.axis_index("c")
    sid = lax.axis_index("s")
    wid = sid * 2 + cid
    lo = NT + wid * RPW
    hi = lo + RPW
    base8 = jnp.minimum((lo // 8) * 8, N - SPAN)

    pltpu.sync_copy(w_hbm, wbuf)
    pltpu.sync_copy(b_hbm, bbuf)

    def idstart(k, carry):
        pltpu.async_copy(seg_hbm.at[pl.ds(base8 + k * C, C)], idbuf.at[k], isem)
        return carry

    lax.fori_loop(0, NCH, idstart, 0)

    def iddrain(k, carry):
        pltpu.make_async_copy(
            seg_hbm.at[pl.ds(base8 + k * C, C)], idbuf.at[k], isem).wait()
        return carry

    lax.fori_loop(0, NCH, iddrain, 0)

    zrows = B // 16
    zero = jnp.zeros((16,), jnp.float32)

    def zbody(r, carry):
        for j in range(NJ):
            sbuf0[r, pl.ds(16 * j, 16)] = zero
        return carry

    lax.fori_loop(0, zrows, zbody, 0)
    pltpu.sync_copy(sbuf0.at[pl.ds(0, zrows)], acc.at[pl.ds(sid * zrows, zrows)])
    plsc.subcore_barrier()

    wv = tuple(wbuf[pl.ds(16 * j, 16)] for j in range(NJ))
    bv = bbuf[...]
    fbufs = (fbuf0, fbuf1)
    sbufs = (sbuf0, sbuf1)
    rsems = (rsem0, rsem1)
    ssems = (ssem0, ssem1)

    def _feats_slice(k):
        return feats_hbm.at[pl.ds(base8 + k * C, C)]

    pltpu.async_copy(_feats_slice(0), fbuf0, rsem0)
    pltpu.async_copy(_feats_slice(1), fbuf1, rsem1)

    def outer(kk, carry):
        for b in (0, 1):
            k = 2 * kk + b
            fb, sb, rs, ss = fbufs[b], sbufs[b], rsems[b], ssems[b]
            pltpu.make_async_copy(_feats_slice(k), fb, rs).wait()

            @pl.when(k >= 2)
            def _wait_prev_scatter():
                pltpu.make_async_copy(sb, acc.at[idbuf.at[k - 2]], ss).wait()

            @plsc.parallel_loop(0, C, unroll=2)
            def row_body(r):
                g = base8 + k * C + r
                valid = jnp.logical_and(g >= lo, g < hi)
                f = [fb[r, pl.ds(16 * j, 16)] for j in range(NJ)]
                p = f[0] * wv[0]
                for j in range(1, NJ):
                    p = p + f[j] * wv[j]
                sv = _allsum16(p) + bv
                m = jnp.where(valid, 1.0, 0.0).astype(jnp.float32)
                w16 = jnp.full((16,), m, jnp.float32) / (1.0 + jnp.exp(-sv))
                for j in range(NJ):
                    sb[r, pl.ds(16 * j, 16)] = f[j] * w16

            @pl.when(k + 2 < NCH)
            def _prefetch_next():
                pltpu.async_copy(_feats_slice(k + 2), fb, rs)

            pltpu.async_copy(sb, acc.at[idbuf.at[k]], ss, add=True)
        return carry

    lax.fori_loop(0, NCH // 2, outer, 0)

    pltpu.make_async_copy(sbuf0, acc.at[idbuf.at[NCH - 2]], ssem0).wait()
    pltpu.make_async_copy(sbuf1, acc.at[idbuf.at[NCH - 1]], ssem1).wait()

    plsc.subcore_barrier()
    pltpu.sync_copy(acc.at[pl.ds(sid * zrows, zrows)],
                    out_hbm.at[cid, pl.ds(sid * zrows, zrows)])


WK = 128


def _tc_body(seg_ref, feats_ref, W_ref, b_ref, out_ref):
    i = pl.program_id(0)

    @pl.when(i == 0)
    def _init():
        out_ref[...] = jnp.zeros_like(out_ref)

    f = feats_ref[...]
    y = jnp.dot(f, W_ref[...], preferred_element_type=jnp.float32) + b_ref[0, 0]
    w = 1.0 / (1.0 + jnp.exp(-y))
    weighted = (f * w).astype(jnp.bfloat16)
    seg = seg_ref[0, 0, :]
    base = jnp.minimum((seg[0] // 8) * 8, B - WK)
    span_ok = seg[R - 1] - base < WK

    @pl.when(span_ok)
    def _windowed():
        onehot = (
            seg[None, :] - base
            == jax.lax.broadcasted_iota(jnp.int32, (WK, R), 0)
        ).astype(jnp.bfloat16)
        out_ref[pl.ds(base, WK), :] += jnp.dot(
            onehot, weighted, preferred_element_type=jnp.float32)

    @pl.when(jnp.logical_not(span_ok))
    def _full():
        onehot = (
            seg[None, :] == jax.lax.broadcasted_iota(jnp.int32, (B, R), 0)
        ).astype(jnp.bfloat16)
        out_ref[...] += jnp.dot(
            onehot, weighted, preferred_element_type=jnp.float32)


def _tc_partial(feats, seg, W, b):
    seg3 = seg.reshape(N // R, 1, R)
    b2 = b.reshape(1, 1).astype(jnp.float32)
    return pl.pallas_call(
        _tc_body,
        grid=(NBT,),
        in_specs=[
            pl.BlockSpec((1, 1, R), lambda i: (i, 0, 0)),
            pl.BlockSpec((R, D), lambda i: (i, 0)),
            pl.BlockSpec((D, 1), lambda i: (0, 0)),
            pl.BlockSpec((1, 1), lambda i: (0, 0)),
        ],
        out_specs=pl.BlockSpec((B, D), lambda i: (0, 0)),
        out_shape=jax.ShapeDtypeStruct((B, D), jnp.float32),
    )(seg3, feats, W, b2)


def _combine_body(tc_ref, p_ref, out_ref):
    out_ref[...] = tc_ref[...] + p_ref[0] + p_ref[1]


def _combine(tc_part, sc_partials):
    return pl.pallas_call(
        _combine_body,
        out_shape=jax.ShapeDtypeStruct((B, D), jnp.float32),
    )(tc_part, sc_partials)


def kernel(feats, segment_ids, W, b):
    seg = segment_ids.astype(jnp.int32)
    bvec = jnp.broadcast_to(b.astype(jnp.float32), (16,))
    sc_partials = _sc_weight_sum(feats, seg, W.reshape(D), bvec)
    tc_part = _tc_partial(feats, seg, W, b)
    return _combine(tc_part, sc_partials)
